# baseline (speedup 1.0000x reference)
import functools

import jax
import jax.numpy as jnp
from jax import lax
from jax.experimental import pallas as pl
from jax.experimental.pallas import tpu as pltpu
from jax.experimental.pallas import tpu_sc as plsc

BATCH = 16384
EMBED_K = 16
ROWS_PER_SUPER = 8
SUPER_W = EMBED_K * ROWS_PER_SUPER
NUM_CORES = 2
NUM_SUBCORES = 16
NUM_WORKERS = NUM_CORES * NUM_SUBCORES
BPW = BATCH // NUM_WORKERS
NCHUNK = 4
CHUNK = BPW // NCHUNK
BLKS = CHUNK // 16


def _build():
    mesh = plsc.VectorSubcoreMesh(core_axis_name="c", subcore_axis_name="s")

    @functools.partial(
        pl.kernel,
        mesh=mesh,
        compiler_params=pltpu.CompilerParams(needs_layout_passes=False),
        out_type=jax.ShapeDtypeStruct((BATCH,), jnp.float32),
        scratch_types=[
            pltpu.VMEM((NCHUNK, CHUNK), jnp.int32),
            pltpu.VMEM((NCHUNK, CHUNK), jnp.int32),
            pltpu.VMEM((CHUNK,), jnp.int32),
            pltpu.VMEM((CHUNK,), jnp.int32),
            pltpu.VMEM((CHUNK, SUPER_W), jnp.float32),
            pltpu.VMEM((CHUNK, SUPER_W), jnp.float32),
            pltpu.VMEM((BPW,), jnp.float32),
            pltpu.SemaphoreType.DMA,
        ],
    )
    def body(xu_hbm, xi_hbm, w_hbm, h_hbm, out_hbm,
             iu, ii, su, si, u, v, o, sem):
        wid = lax.axis_index("s") * NUM_CORES + lax.axis_index("c")

        pltpu.sync_copy(xu_hbm.at[wid], iu)
        pltpu.sync_copy(xi_hbm.at[wid], ii)

        lane = lax.iota(jnp.int32, 16)

        for c in range(NCHUNK):
            for t in range(CHUNK // 16):
                sl = pl.ds(t * 16, 16)
                su[sl] = lax.shift_right_logical(iu[c, sl], 3)
                si[sl] = lax.shift_right_logical(ii[c, sl], 3)
            cu = pltpu.async_copy(w_hbm.at[su], u, sem)
            cv = pltpu.async_copy(h_hbm.at[si], v, sem)
            cu.wait()
            cv.wait()

            for b in range(BLKS):
                sl = pl.ds(b * 16, 16)
                rows = b * 16 + lane
                ju = (iu[c, sl] & 7) * EMBED_K
                jv = (ii[c, sl] & 7) * EMBED_K
                acc = plsc.load_gather(u, [rows, ju]) * plsc.load_gather(
                    v, [rows, jv])
                for k in range(1, EMBED_K):
                    acc = acc + plsc.load_gather(u, [rows, ju + k]) * \
                        plsc.load_gather(v, [rows, jv + k])
                o[pl.ds(c * CHUNK + b * 16, 16)] = 1.0 / (1.0 + jnp.exp(-acc))

        pltpu.sync_copy(o, out_hbm.at[pl.ds(wid * BPW, BPW)])

    return body


_KERNEL = _build()


def kernel(x, W, H):
    xu = x[:, 0].reshape(NUM_WORKERS, NCHUNK, CHUNK)
    xi = x[:, 1].reshape(NUM_WORKERS, NCHUNK, CHUNK)
    w_view = W.reshape(W.shape[0] // ROWS_PER_SUPER, SUPER_W)
    h_view = H.reshape(H.shape[0] // ROWS_PER_SUPER, SUPER_W)
    return _KERNEL(xu, xi, w_view, h_view)


# speedup vs baseline: 6.1683x; 6.1683x over previous
import functools

import jax
import jax.numpy as jnp
from jax import lax
from jax.experimental import pallas as pl
from jax.experimental.pallas import tpu as pltpu
from jax.experimental.pallas import tpu_sc as plsc

BATCH = 16384
EMBED_K = 16
NUM_ROWS = 1000000
NUM_CORES = 2
NUM_SUBCORES = 16
NUM_WORKERS = NUM_CORES * NUM_SUBCORES
BPW = BATCH // NUM_WORKERS
NBLK = BPW // 16


def _build():
    mesh = plsc.VectorSubcoreMesh(core_axis_name="c", subcore_axis_name="s")

    @functools.partial(
        pl.kernel,
        mesh=mesh,
        compiler_params=pltpu.CompilerParams(needs_layout_passes=False),
        out_type=jax.ShapeDtypeStruct((BATCH,), jnp.float32),
        scratch_types=[
            pltpu.VMEM((NBLK, 16), jnp.int32),
            pltpu.VMEM((NBLK, 16), jnp.int32),
            pltpu.VMEM((16, EMBED_K, 128), jnp.float32),
            pltpu.VMEM((16, EMBED_K, 128), jnp.float32),
            pltpu.VMEM((BPW,), jnp.float32),
            pltpu.SemaphoreType.DMA,
        ],
    )
    def body(xu_hbm, xi_hbm, wt_hbm, ht_hbm, out_hbm, iu, ii, u, v, o, sem):
        wid = lax.axis_index("s") * NUM_CORES + lax.axis_index("c")

        pltpu.sync_copy(xu_hbm.at[wid], iu)
        pltpu.sync_copy(xi_hbm.at[wid], ii)

        lane = lax.iota(jnp.int32, 16)
        kvecs = [jnp.full((16,), k, jnp.int32) for k in range(EMBED_K)]

        def blk(b, carry):
            bu = iu[b]
            bv = ii[b]
            cu = bu & 127
            cv = bv & 127
            tu = lax.shift_right_logical(bu, 7) * 128
            tv = lax.shift_right_logical(bv, 7) * 128
            copies = []
            for j in range(16):
                src_u = wt_hbm.at[:, pl.ds(pl.multiple_of(tu[j], 128), 128)]
                src_v = ht_hbm.at[:, pl.ds(pl.multiple_of(tv[j], 128), 128)]
                copies.append(pltpu.async_copy(src_u, u.at[j], sem))
                copies.append(pltpu.async_copy(src_v, v.at[j], sem))
            for cp in copies:
                cp.wait()

            acc = plsc.load_gather(u, [lane, kvecs[0], cu]) * \
                plsc.load_gather(v, [lane, kvecs[0], cv])
            for k in range(1, EMBED_K):
                acc = acc + plsc.load_gather(u, [lane, kvecs[k], cu]) * \
                    plsc.load_gather(v, [lane, kvecs[k], cv])
            o[pl.ds(b * 16, 16)] = 1.0 / (1.0 + jnp.exp(-acc))
            return carry

        lax.fori_loop(0, NBLK, blk, 0)

        pltpu.sync_copy(o, out_hbm.at[pl.ds(wid * BPW, BPW)])

    return body


_KERNEL = _build()


def kernel(x, W, H):
    xu = x[:, 0].reshape(NUM_WORKERS, NBLK, 16)
    xi = x[:, 1].reshape(NUM_WORKERS, NBLK, 16)
    return _KERNEL(xu, xi, W.T, H.T)
